# baseline (speedup 1.0000x reference)
import functools

import jax
import jax.numpy as jnp
from jax import lax
from jax.experimental import pallas as pl
from jax.experimental.pallas import tpu as pltpu
from jax.experimental.pallas import tpu_sc as plsc

F = 32
V = 1_000_000
VMAIN = V - V % 128
NPACK = V // 4
NCH1 = VMAIN // 128
NC = 2
NS = 16
NW = NC * NS
NB = 4
TRIP1 = 248

TOK = 16384 * 50
TPW = TOK // NW
HALF = 256
NB2 = 2
NCH2 = 100


def _iota16():
  return lax.iota(jnp.int32, 16)


def _c1_body(wt_hbm, tailp_hbm, rt_hbm, *refs):
  wid = lax.axis_index("s") * NC + lax.axis_index("c")
  inbuf = refs[0:NB]
  packbuf = refs[NB:2 * NB]
  isem = refs[2 * NB:3 * NB]
  osem = refs[3 * NB:4 * NB]
  iota = _iota16()

  def cidx(i):
    return jnp.minimum(wid + NW * i, NCH1 - 1)

  def fire_in(i, b):
    pltpu.async_copy(
        wt_hbm.at[pl.ds(0, F), pl.ds(cidx(i) * 128, 128)], inbuf[b], isem[b])

  @pl.when(wid == 0)
  def _():
    pltpu.sync_copy(tailp_hbm, refs[0].at[pl.ds(0, 16)])
    pltpu.sync_copy(refs[0].at[pl.ds(0, 16)], rt_hbm.at[pl.ds(NPACK - 16, 16)])

  for b in range(NB):
    fire_in(b, b)

  def chunk(i, b, first):
    pltpu.make_async_copy(
        wt_hbm.at[pl.ds(0, F), pl.ds(0, 128)], inbuf[b], isem[b]).wait()
    if not first:
      pltpu.make_async_copy(
          packbuf[b], rt_hbm.at[pl.ds(0, F)], osem[b]).wait()
    row0 = iota
    row1 = 16 + iota
    zeros = jnp.zeros((16,), jnp.int32)
    @plsc.parallel_loop(0, 32, unroll=8)
    def _(q):
      q4 = 4 * q + zeros
      cols = [q4, q4 + 1, q4 + 2, q4 + 3]
      for j in range(8):
        packbuf[b][q, pl.ds(16 * j, 16)] = plsc.load_gather(
            inbuf[b], [row1 if j & 1 else row0, cols[j // 2]])
    fire_in(i + NB, b)
    pltpu.async_copy(
        packbuf[b], rt_hbm.at[pl.ds(cidx(i) * F, F)], osem[b])

  for b in range(NB):
    chunk(b, b, True)

  def body(j, carry):
    for b in range(NB):
      chunk(NB * j + b, b, False)
    return carry

  lax.fori_loop(1, TRIP1 // NB, body, 0)

  for b in range(NB):
    pltpu.make_async_copy(
        packbuf[b], rt_hbm.at[pl.ds(0, F)], osem[b]).wait()
    pltpu.make_async_copy(
        wt_hbm.at[pl.ds(0, F), pl.ds(0, 128)], inbuf[b], isem[b]).wait()


def _c2_body(xf_hbm, rt_hbm, q_hbm, idxbuf, *refs):
  wid = lax.axis_index("s") * NC + lax.axis_index("c")
  tb = wid * (TPW // 50)
  plist = refs[0:NB2]
  colb = refs[NB2:2 * NB2]
  rowsb = refs[2 * NB2:3 * NB2]
  qbufs = refs[3 * NB2:5 * NB2]
  qbuf = (qbufs[0:2], qbufs[2:4])
  gsem = refs[5 * NB2:6 * NB2]
  osem = refs[6 * NB2:7 * NB2]
  iota = _iota16()

  pltpu.sync_copy(xf_hbm.at[pl.ds(wid * TPW, TPW)], idxbuf)

  def fire_chunk(n, b):
    s = n // 2
    h = n % 2
    @plsc.parallel_loop(0, HALF // 16, unroll=8)
    def _(k):
      pos = (12800 * h + s + 800 * k) + iota * 50
      v = plsc.load_gather(idxbuf, [pos])
      plist[b][k // 8, pl.ds(16 * (k % 8), 16)] = v >> 2
      colb[b][pl.ds(16 * k, 16)] = (v & 3) << 5
    pltpu.async_copy(
        rt_hbm.at[plist[b].at[0]], rowsb[b].at[pl.ds(0, 128)], gsem[b])
    pltpu.async_copy(
        rt_hbm.at[plist[b].at[1]], rowsb[b].at[pl.ds(128, 128)], gsem[b])

  for b in range(NB2):
    fire_chunk(b, b)

  def chunk(n, b, first):
    s = n // 2
    h = n % 2
    pltpu.make_async_copy(
        rt_hbm.at[pl.ds(0, HALF)], rowsb[b], gsem[b]).wait()
    if not first:
      for u in range(2):
        pltpu.make_async_copy(
            qbuf[b][u], q_hbm.at[0, pl.ds(0, F), pl.ds(0, 128)],
            osem[b]).wait()

    for u in range(2):
      qb = qbuf[b][u]
      @plsc.parallel_loop(8 * u, 8 * u + 8, unroll=4)
      def _(g):
        rv = 16 * g + iota
        cb = colb[b][pl.ds(16 * g, 16)]
        for f in range(F):
          qb[f, pl.ds(16 * g - 128 * u, 16)] = plsc.load_gather(
              rowsb[b], [rv, cb + f])

    @pl.when(n + NB2 < NCH2)
    def _():
      fire_chunk(n + NB2, b)

    for u in range(2):
      pltpu.async_copy(
          qbuf[b][u],
          q_hbm.at[s, pl.ds(0, F), pl.ds(tb + HALF * h + 128 * u, 128)],
          osem[b])

  for b in range(NB2):
    chunk(b, b, True)

  def body(j, carry):
    for b in range(NB2):
      chunk(NB2 * j + b, b, False)
    return carry

  lax.fori_loop(1, NCH2 // NB2, body, 0)

  for b in range(NB2):
    for u in range(2):
      pltpu.make_async_copy(
          qbuf[b][u], q_hbm.at[0, pl.ds(0, F), pl.ds(0, 128)],
          osem[b]).wait()


@functools.lru_cache(maxsize=None)
def _build():
  mesh = plsc.VectorSubcoreMesh(core_axis_name="c", subcore_axis_name="s")
  params = pltpu.CompilerParams(
      use_tc_tiling_on_sc=True, needs_layout_passes=False)
  c1 = pl.kernel(
      _c1_body,
      out_type=jax.ShapeDtypeStruct((NPACK, 128), jnp.float32),
      mesh=mesh,
      compiler_params=params,
      scratch_types=(
          [pltpu.VMEM((F, 128), jnp.float32)] * (2 * NB)
          + [pltpu.SemaphoreType.DMA] * (2 * NB)
      ),
  )
  c2 = pl.kernel(
      _c2_body,
      out_type=jax.ShapeDtypeStruct((50, F, 16384), jnp.float32),
      mesh=mesh,
      compiler_params=params,
      scratch_types=(
          [pltpu.VMEM((TPW,), jnp.int32)]
          + [pltpu.VMEM((2, 128), jnp.int32)] * NB2
          + [pltpu.VMEM((HALF,), jnp.int32)] * NB2
          + [pltpu.VMEM((HALF, 128), jnp.float32)] * NB2
          + [pltpu.VMEM((F, 128), jnp.float32)] * (2 * NB2)
          + [pltpu.SemaphoreType.DMA] * (2 * NB2)
      ),
  )
  return c1, c2


def kernel(x, weight):
  c1, c2 = _build()
  xf = x.reshape(-1)
  w_t = weight.T
  tailp = weight[VMAIN:, :].reshape(16, 128)
  rowtable = c1(w_t, tailp)
  q = c2(xf, rowtable)
  return q.transpose(2, 0, 1)


# speedup vs baseline: 1.0022x; 1.0022x over previous
import functools

import jax
import jax.numpy as jnp
from jax import lax
from jax.experimental import pallas as pl
from jax.experimental.pallas import tpu as pltpu
from jax.experimental.pallas import tpu_sc as plsc

F = 32
V = 1_000_000
VMAIN = V - V % 128
NPACK = V // 4
NCH1 = VMAIN // 128
NC = 2
NS = 16
NW = NC * NS
NB = 4
TRIP1 = 248

TOK = 16384 * 50
TPW = TOK // NW
HALF = 256
NB2 = 2
NCH2 = 100


def _iota16():
  return lax.iota(jnp.int32, 16)


def _c1_body(wt_hbm, tailp_hbm, rt_hbm, *refs):
  wid = lax.axis_index("s") * NC + lax.axis_index("c")
  inbuf = refs[0:NB]
  packbuf = refs[NB:2 * NB]
  isem = refs[2 * NB:3 * NB]
  osem = refs[3 * NB:4 * NB]
  iota = _iota16()

  def cidx(i):
    return jnp.minimum(wid + NW * i, NCH1 - 1)

  def fire_in(i, b):
    pltpu.async_copy(
        wt_hbm.at[pl.ds(0, F), pl.ds(cidx(i) * 128, 128)], inbuf[b], isem[b])

  @pl.when(wid == 0)
  def _():
    pltpu.sync_copy(tailp_hbm, refs[0].at[pl.ds(0, 16)])
    pltpu.sync_copy(refs[0].at[pl.ds(0, 16)], rt_hbm.at[pl.ds(NPACK - 16, 16)])

  for b in range(NB):
    fire_in(b, b)

  def chunk(i, b, first):
    pltpu.make_async_copy(
        wt_hbm.at[pl.ds(0, F), pl.ds(0, 128)], inbuf[b], isem[b]).wait()
    if not first:
      pltpu.make_async_copy(
          packbuf[b], rt_hbm.at[pl.ds(0, F)], osem[b]).wait()
    row0 = iota
    row1 = 16 + iota
    zeros = jnp.zeros((16,), jnp.int32)
    @plsc.parallel_loop(0, 32, unroll=4)
    def _(q):
      q4 = 4 * q + zeros
      cols = [q4, q4 + 1, q4 + 2, q4 + 3]
      for j in range(8):
        packbuf[b][q, pl.ds(16 * j, 16)] = plsc.load_gather(
            inbuf[b], [row1 if j & 1 else row0, cols[j // 2]])
    fire_in(i + NB, b)
    pltpu.async_copy(
        packbuf[b], rt_hbm.at[pl.ds(cidx(i) * F, F)], osem[b])

  for b in range(NB):
    chunk(b, b, True)

  def body(j, carry):
    for b in range(NB):
      chunk(NB * j + b, b, False)
    return carry

  lax.fori_loop(1, TRIP1 // NB, body, 0)

  for b in range(NB):
    pltpu.make_async_copy(
        packbuf[b], rt_hbm.at[pl.ds(0, F)], osem[b]).wait()
    pltpu.make_async_copy(
        wt_hbm.at[pl.ds(0, F), pl.ds(0, 128)], inbuf[b], isem[b]).wait()


def _c2_body(xf_hbm, rt_hbm, q_hbm, idxbuf, *refs):
  wid = lax.axis_index("s") * NC + lax.axis_index("c")
  tb = wid * (TPW // 50)
  plist = refs[0:NB2]
  colb = refs[NB2:2 * NB2]
  rowsb = refs[2 * NB2:3 * NB2]
  qbufs = refs[3 * NB2:5 * NB2]
  qbuf = (qbufs[0:2], qbufs[2:4])
  gsem = refs[5 * NB2:6 * NB2]
  osem = refs[6 * NB2:7 * NB2]
  iota = _iota16()

  pltpu.sync_copy(xf_hbm.at[pl.ds(wid * TPW, TPW)], idxbuf)

  def fire_chunk(n, b):
    s = n // 2
    h = n % 2
    @plsc.parallel_loop(0, HALF // 16, unroll=8)
    def _(k):
      pos = (12800 * h + s + 800 * k) + iota * 50
      v = plsc.load_gather(idxbuf, [pos])
      plist[b][k // 8, pl.ds(16 * (k % 8), 16)] = v >> 2
      colb[b][pl.ds(16 * k, 16)] = (v & 3) << 5
    pltpu.async_copy(
        rt_hbm.at[plist[b].at[0]], rowsb[b].at[pl.ds(0, 128)], gsem[b])
    pltpu.async_copy(
        rt_hbm.at[plist[b].at[1]], rowsb[b].at[pl.ds(128, 128)], gsem[b])

  for b in range(NB2):
    fire_chunk(b, b)

  def chunk(n, b, first):
    s = n // 2
    h = n % 2
    pltpu.make_async_copy(
        rt_hbm.at[pl.ds(0, HALF)], rowsb[b], gsem[b]).wait()
    if not first:
      for u in range(2):
        pltpu.make_async_copy(
            qbuf[b][u], q_hbm.at[0, pl.ds(0, F), pl.ds(0, 128)],
            osem[b]).wait()

    for u in range(2):
      qb = qbuf[b][u]
      @plsc.parallel_loop(8 * u, 8 * u + 8, unroll=4)
      def _(g):
        rv = 16 * g + iota
        cb = colb[b][pl.ds(16 * g, 16)]
        for f in range(F):
          qb[f, pl.ds(16 * g - 128 * u, 16)] = plsc.load_gather(
              rowsb[b], [rv, cb + f])

    @pl.when(n + NB2 < NCH2)
    def _():
      fire_chunk(n + NB2, b)

    for u in range(2):
      pltpu.async_copy(
          qbuf[b][u],
          q_hbm.at[s, pl.ds(0, F), pl.ds(tb + HALF * h + 128 * u, 128)],
          osem[b])

  for b in range(NB2):
    chunk(b, b, True)

  def body(j, carry):
    for b in range(NB2):
      chunk(NB2 * j + b, b, False)
    return carry

  lax.fori_loop(1, NCH2 // NB2, body, 0)

  for b in range(NB2):
    for u in range(2):
      pltpu.make_async_copy(
          qbuf[b][u], q_hbm.at[0, pl.ds(0, F), pl.ds(0, 128)],
          osem[b]).wait()


@functools.lru_cache(maxsize=None)
def _build():
  mesh = plsc.VectorSubcoreMesh(core_axis_name="c", subcore_axis_name="s")
  params = pltpu.CompilerParams(
      use_tc_tiling_on_sc=True, needs_layout_passes=False)
  c1 = pl.kernel(
      _c1_body,
      out_type=jax.ShapeDtypeStruct((NPACK, 128), jnp.float32),
      mesh=mesh,
      compiler_params=params,
      scratch_types=(
          [pltpu.VMEM((F, 128), jnp.float32)] * (2 * NB)
          + [pltpu.SemaphoreType.DMA] * (2 * NB)
      ),
  )
  c2 = pl.kernel(
      _c2_body,
      out_type=jax.ShapeDtypeStruct((50, F, 16384), jnp.float32),
      mesh=mesh,
      compiler_params=params,
      scratch_types=(
          [pltpu.VMEM((TPW,), jnp.int32)]
          + [pltpu.VMEM((2, 128), jnp.int32)] * NB2
          + [pltpu.VMEM((HALF,), jnp.int32)] * NB2
          + [pltpu.VMEM((HALF, 128), jnp.float32)] * NB2
          + [pltpu.VMEM((F, 128), jnp.float32)] * (2 * NB2)
          + [pltpu.SemaphoreType.DMA] * (2 * NB2)
      ),
  )
  return c1, c2


def kernel(x, weight):
  c1, c2 = _build()
  xf = x.reshape(-1)
  w_t = weight.T
  tailp = weight[VMAIN:, :].reshape(16, 128)
  rowtable = c1(w_t, tailp)
  q = c2(xf, rowtable)
  return q.transpose(2, 0, 1)


# speedup vs baseline: 1.0644x; 1.0620x over previous
import functools

import jax
import jax.numpy as jnp
from jax import lax
from jax.experimental import pallas as pl
from jax.experimental.pallas import tpu as pltpu
from jax.experimental.pallas import tpu_sc as plsc

F = 32
V = 1_000_000
VMAIN = V - V % 128
NPACK = V // 4
NCH1 = VMAIN // 128
NC = 2
NS = 16
NW = NC * NS
NB = 4
TRIP1 = 248

TOK = 16384 * 50
TPW = TOK // NW
HALF = 256
NB2 = 2
NCH2 = 100


def _iota16():
  return lax.iota(jnp.int32, 16)


def _c1_body(wt_hbm, tailp_hbm, rt_hbm, *refs):
  wid = lax.axis_index("s") * NC + lax.axis_index("c")
  inbuf = refs[0:NB]
  packbuf = refs[NB:2 * NB]
  isem = refs[2 * NB:3 * NB]
  osem = refs[3 * NB:4 * NB]
  iota = _iota16()

  def cidx(i):
    return jnp.minimum(wid + NW * i, NCH1 - 1)

  def fire_in(i, b):
    pltpu.async_copy(
        wt_hbm.at[pl.ds(0, F), pl.ds(cidx(i) * 128, 128)], inbuf[b], isem[b])

  @pl.when(wid == 0)
  def _():
    pltpu.sync_copy(tailp_hbm, refs[0].at[pl.ds(0, 16)])
    pltpu.sync_copy(refs[0].at[pl.ds(0, 16)], rt_hbm.at[pl.ds(NPACK - 16, 16)])

  for b in range(NB):
    fire_in(b, b)

  def chunk(i, b, first):
    pltpu.make_async_copy(
        wt_hbm.at[pl.ds(0, F), pl.ds(0, 128)], inbuf[b], isem[b]).wait()
    if not first:
      pltpu.make_async_copy(
          packbuf[b], rt_hbm.at[pl.ds(0, F)], osem[b]).wait()
    row0 = iota
    row1 = 16 + iota
    zeros = jnp.zeros((16,), jnp.int32)
    @plsc.parallel_loop(0, 32, unroll=4)
    def _(q):
      q4 = 4 * q + zeros
      for j in range(8):
        packbuf[b][q, pl.ds(16 * j, 16)] = plsc.load_gather(
            inbuf[b], [row1 if j & 1 else row0, q4 + (j // 2)])
    fire_in(i + NB, b)
    pltpu.async_copy(
        packbuf[b], rt_hbm.at[pl.ds(cidx(i) * F, F)], osem[b])

  for b in range(NB):
    chunk(b, b, True)

  def body(j, carry):
    for b in range(NB):
      chunk(NB * j + b, b, False)
    return carry

  lax.fori_loop(1, TRIP1 // NB, body, 0)

  for b in range(NB):
    pltpu.make_async_copy(
        packbuf[b], rt_hbm.at[pl.ds(0, F)], osem[b]).wait()
    pltpu.make_async_copy(
        wt_hbm.at[pl.ds(0, F), pl.ds(0, 128)], inbuf[b], isem[b]).wait()


def _c2_body(xf_hbm, rt_hbm, q_hbm, idxbuf, *refs):
  wid = lax.axis_index("s") * NC + lax.axis_index("c")
  tb = wid * (TPW // 50)
  plist = refs[0:NB2]
  colb = refs[NB2:2 * NB2]
  rowsb = refs[2 * NB2:3 * NB2]
  qbufs = refs[3 * NB2:5 * NB2]
  qbuf = (qbufs[0:2], qbufs[2:4])
  gsem = refs[5 * NB2:6 * NB2]
  osem = refs[6 * NB2:7 * NB2]
  iota = _iota16()

  pltpu.sync_copy(xf_hbm.at[pl.ds(wid * TPW, TPW)], idxbuf)

  def fire_chunk(n, b):
    s = n // 2
    h = n % 2
    @plsc.parallel_loop(0, HALF // 16, unroll=8)
    def _(k):
      pos = (12800 * h + s + 800 * k) + iota * 50
      v = plsc.load_gather(idxbuf, [pos])
      plist[b][k // 8, pl.ds(16 * (k % 8), 16)] = v >> 2
      colb[b][pl.ds(16 * k, 16)] = (v & 3) << 5
    pltpu.async_copy(
        rt_hbm.at[plist[b].at[0]], rowsb[b].at[pl.ds(0, 128)], gsem[b])
    pltpu.async_copy(
        rt_hbm.at[plist[b].at[1]], rowsb[b].at[pl.ds(128, 128)], gsem[b])

  for b in range(NB2):
    fire_chunk(b, b)

  def chunk(n, b, first):
    s = n // 2
    h = n % 2
    pltpu.make_async_copy(
        rt_hbm.at[pl.ds(0, HALF)], rowsb[b], gsem[b]).wait()
    if not first:
      for u in range(2):
        pltpu.make_async_copy(
            qbuf[b][u], q_hbm.at[0, pl.ds(0, F), pl.ds(0, 128)],
            osem[b]).wait()

    for u in range(2):
      qb = qbuf[b][u]
      @plsc.parallel_loop(8 * u, 8 * u + 8, unroll=2)
      def _(g):
        rv = 16 * g + iota
        cb = colb[b][pl.ds(16 * g, 16)]
        for f in range(F):
          qb[f, pl.ds(16 * g - 128 * u, 16)] = plsc.load_gather(
              rowsb[b], [rv, cb + f])

    @pl.when(n + NB2 < NCH2)
    def _():
      fire_chunk(n + NB2, b)

    for u in range(2):
      pltpu.async_copy(
          qbuf[b][u],
          q_hbm.at[s, pl.ds(0, F), pl.ds(tb + HALF * h + 128 * u, 128)],
          osem[b])

  for b in range(NB2):
    chunk(b, b, True)

  def body(j, carry):
    for b in range(NB2):
      chunk(NB2 * j + b, b, False)
    return carry

  lax.fori_loop(1, NCH2 // NB2, body, 0)

  for b in range(NB2):
    for u in range(2):
      pltpu.make_async_copy(
          qbuf[b][u], q_hbm.at[0, pl.ds(0, F), pl.ds(0, 128)],
          osem[b]).wait()


@functools.lru_cache(maxsize=None)
def _build():
  mesh = plsc.VectorSubcoreMesh(core_axis_name="c", subcore_axis_name="s")
  params = pltpu.CompilerParams(
      use_tc_tiling_on_sc=True, needs_layout_passes=False)
  c1 = pl.kernel(
      _c1_body,
      out_type=jax.ShapeDtypeStruct((NPACK, 128), jnp.float32),
      mesh=mesh,
      compiler_params=params,
      scratch_types=(
          [pltpu.VMEM((F, 128), jnp.float32)] * (2 * NB)
          + [pltpu.SemaphoreType.DMA] * (2 * NB)
      ),
  )
  c2 = pl.kernel(
      _c2_body,
      out_type=jax.ShapeDtypeStruct((50, F, 16384), jnp.float32),
      mesh=mesh,
      compiler_params=params,
      scratch_types=(
          [pltpu.VMEM((TPW,), jnp.int32)]
          + [pltpu.VMEM((2, 128), jnp.int32)] * NB2
          + [pltpu.VMEM((HALF,), jnp.int32)] * NB2
          + [pltpu.VMEM((HALF, 128), jnp.float32)] * NB2
          + [pltpu.VMEM((F, 128), jnp.float32)] * (2 * NB2)
          + [pltpu.SemaphoreType.DMA] * (2 * NB2)
      ),
  )
  return c1, c2


def kernel(x, weight):
  c1, c2 = _build()
  xf = x.reshape(-1)
  w_t = weight.T
  tailp = weight[VMAIN:, :].reshape(16, 128)
  rowtable = c1(w_t, tailp)
  q = c2(xf, rowtable)
  return q.transpose(2, 0, 1)
